# baseline (speedup 1.0000x reference)
import functools

import jax
import jax.numpy as jnp
from jax import lax
from jax.experimental import pallas as pl
from jax.experimental.pallas import tpu as pltpu
from jax.experimental.pallas import tpu_sc as plsc

_N = 20000
_G = 20
_NR = _N + _G
_ROWS = 160
_NP = _ROWS * 128
_NCH = _NP // 16
_NL2 = _NCH // 16
_K = 128
_KFG = 32
_B = 4
_STD = (0.1, 0.1, 0.2, 0.2)


def _first(mask, lane):
    return jnp.min(jnp.where(mask, lane, 10_000))


def _sc_body(coords_hbm, gt_hbm, out_hbm,
             x1_v, y1_v, x2_v, y2_v, psl_v, gt_v, sp_prio,
             prio_v, cmax_v, l2_v, keep_v, keepm_v):
    cid = lax.axis_index("c")
    sid = lax.axis_index("s")
    lane = lax.broadcasted_iota(jnp.int32, (16,), 0)

    bloc = sid // 8
    b = cid * 2 + bloc
    sl = sid % 8
    off = sl * (_NP // 8)

    pltpu.sync_copy(gt_hbm.at[b], gt_v)
    gl = []
    gh = []
    for i in range(6):
        gl.append(gt_v[pl.ds(i * 32, 16)])
        gh.append(gt_v[pl.ds(i * 32 + 16, 16)])

    def _sca(vecs, g):
        v = vecs[0] if g < 16 else vecs[1]
        return jnp.sum(jnp.where(lane == (g % 16), v, 0.0))

    nb = _sca((gl[5], gh[5]), 0)
    gxs = [( _sca((gl[0], gh[0]), g), _sca((gl[1], gh[1]), g),
             _sca((gl[2], gh[2]), g), _sca((gl[3], gh[3]), g))
           for g in range(_G)]

    pltpu.sync_copy(coords_hbm.at[b, 0, pl.ds(off, _NP // 8)], x1_v)
    pltpu.sync_copy(coords_hbm.at[b, 1, pl.ds(off, _NP // 8)], y1_v)
    pltpu.sync_copy(coords_hbm.at[b, 2, pl.ds(off, _NP // 8)], x2_v)
    pltpu.sync_copy(coords_hbm.at[b, 3, pl.ds(off, _NP // 8)], y2_v)

    def chunk(k, _):
        x1 = x1_v[pl.ds(k * 16, 16)]
        y1 = y1_v[pl.ds(k * 16, 16)]
        x2 = x2_v[pl.ds(k * 16, 16)]
        y2 = y2_v[pl.ds(k * 16, 16)]
        area = (x2 - x1 + 1.0) * (y2 - y1 + 1.0)
        run_max = jnp.full((16,), -2.0, jnp.float32)
        for g in range(_G):
            gx1, gy1, gx2, gy2 = gxs[g]
            iw = jnp.clip(jnp.minimum(x2, gx2) - jnp.maximum(x1, gx1) + 1.0,
                          0.0)
            ih = jnp.clip(jnp.minimum(y2, gy2) - jnp.maximum(y1, gy1) + 1.0,
                          0.0)
            inter = iw * ih
            garea = (gx2 - gx1 + 1.0) * (gy2 - gy1 + 1.0)
            iou = inter / (area + garea - inter + 1e-6)
            val = jnp.where(jnp.float32(g) < nb, iou, -1.0)
            run_max = jnp.maximum(run_max, val)
        fg = run_max >= 0.5
        bgm = jnp.logical_and(run_max < 0.5, run_max >= 0.1)
        prio = (run_max + jnp.where(fg, 10.0, 0.0)
                + jnp.where(bgm, 5.0, 0.0))
        gidx = off + k * 16 + lane
        prio = jnp.where(gidx < _NR, prio, -1.0)
        psl_v[pl.ds(k * 16, 16)] = prio
        return 0
    lax.fori_loop(0, _NP // 8 // 16, chunk, 0)

    pltpu.sync_copy(psl_v, sp_prio.at[bloc, pl.ds(off, _NP // 8)])
    plsc.subcore_barrier()

    @pl.when(jnp.logical_or(sid == 0, sid == 8))
    def _():
        b = cid * 2 + sid // 8
        pltpu.sync_copy(sp_prio.at[sid // 8], prio_v)

        def build_cmax(k, _):
            acc = jnp.full((16,), -9.0, jnp.float32)
            for j in range(16):
                v = prio_v[pl.ds((k * 16 + j) * 16, 16)]
                acc = jnp.where(lane == j, jnp.max(v), acc)
            cmax_v[pl.ds(k * 16, 16)] = acc
            return 0
        lax.fori_loop(0, _NL2, build_cmax, 0)

        def build_l2(k, _):
            acc = jnp.full((16,), -9.0, jnp.float32)
            for j in range(16):
                v = cmax_v[pl.ds((k * 16 + j) * 16, 16)]
                acc = jnp.where(lane == j, jnp.max(v), acc)
            l2_v[pl.ds(k * 16, 16)] = acc
            return 0
        lax.fori_loop(0, _NL2 // 16, build_l2, 0)

        l3 = jnp.full((16,), -9.0, jnp.float32)
        for h in range(_NL2 // 16):
            v = l2_v[pl.ds(h * 16, 16)]
            l3 = jnp.where(lane == h, jnp.max(v), l3)

        def outer(o, l3):
            ki = jnp.zeros((16,), jnp.float32)
            km = jnp.zeros((16,), jnp.float32)
            for j in range(16):
                m = jnp.max(l3)
                h = _first(l3 == m, lane)
                l2v = l2_v[pl.ds(h * 16, 16)]
                s2 = h * 16 + _first(l2v == m, lane)
                cmv = cmax_v[pl.ds(s2 * 16, 16)]
                s3 = s2 * 16 + _first(cmv == m, lane)
                pv = prio_v[pl.ds(s3 * 16, 16)]
                s4 = _first(pv == m, lane)
                idx = s3 * 16 + s4
                pv = jnp.where(lane == s4, -3.0, pv)
                prio_v[pl.ds(s3 * 16, 16)] = pv
                cmv = jnp.where(lane == (s3 - s2 * 16), jnp.max(pv), cmv)
                cmax_v[pl.ds(s2 * 16, 16)] = cmv
                l2v = jnp.where(lane == (s2 - h * 16), jnp.max(cmv), l2v)
                l2_v[pl.ds(h * 16, 16)] = l2v
                l3 = jnp.where(lane == h, jnp.max(l2v), l3)
                ki = jnp.where(lane == j, idx.astype(jnp.float32), ki)
                km = jnp.where(lane == j, m, km)
            keep_v[pl.ds(o * 16, 16)] = ki
            keepm_v[pl.ds(o * 16, 16)] = km
            return l3
        lax.fori_loop(0, _K // 16, outer, l3)

        pltpu.sync_copy(keep_v, out_hbm.at[b, 0])
        pltpu.sync_copy(keepm_v, out_hbm.at[b, 1])


_sc_select = functools.partial(
    pl.kernel,
    out_type=jax.ShapeDtypeStruct((_B, 2, _K), jnp.float32),
    mesh=plsc.VectorSubcoreMesh(core_axis_name="c", subcore_axis_name="s"),
    scratch_types=[
        pltpu.VMEM((_NP // 8,), jnp.float32),
        pltpu.VMEM((_NP // 8,), jnp.float32),
        pltpu.VMEM((_NP // 8,), jnp.float32),
        pltpu.VMEM((_NP // 8,), jnp.float32),
        pltpu.VMEM((_NP // 8,), jnp.float32),
        pltpu.VMEM((192,), jnp.float32),
        pltpu.VMEM_SHARED((2, _NP), jnp.float32),
        pltpu.VMEM((_NP,), jnp.float32),
        pltpu.VMEM((_NCH,), jnp.float32),
        pltpu.VMEM((_NL2,), jnp.float32),
        pltpu.VMEM((_K,), jnp.float32),
        pltpu.VMEM((_K,), jnp.float32),
    ],
    compiler_params=pltpu.CompilerParams(needs_layout_passes=False),
)(_sc_body)


def _tc2_body(coords_ref, nb_ref, sel_ref, gt_ref, info_ref, out_ref):
    x1 = coords_ref[:, 0]
    y1 = coords_ref[:, 1]
    x2 = coords_ref[:, 2]
    y2 = coords_ref[:, 3]
    nbv = nb_ref[...]
    gtv = gt_ref[...]
    infov = info_ref[...]

    idx = sel_ref[:, 0:1, :]
    sm = sel_ref[:, 1:2, :]
    kr = jnp.floor(idx * (1.0 / 128.0))
    kc = idx - 128.0 * kr

    lane = jax.lax.broadcasted_iota(
        jnp.int32, (1, 1, 128), 2).astype(jnp.float32)

    e_c = jnp.where(
        jax.lax.broadcasted_iota(
            jnp.int32, (_B, 128, 128), 1).astype(jnp.float32) == kc,
        1.0, 0.0)
    e_r = jnp.where(
        jax.lax.broadcasted_iota(
            jnp.int32, (_B, _ROWS, 128), 1).astype(jnp.float32) == kr,
        1.0, 0.0)

    def pick(q):
        y = jax.lax.dot_general(
            q, e_c, dimension_numbers=(((2,), (1,)), ((0,), (0,))),
            precision=jax.lax.Precision.HIGHEST)
        return jnp.sum(e_r * y, axis=1, keepdims=True)

    sx1 = pick(x1)
    sy1 = pick(y1)
    sx2 = pick(x2)
    sy2 = pick(y2)

    sarea = (sx2 - sx1 + 1.0) * (sy2 - sy1 + 1.0)
    run_max = jnp.full(sx1.shape, -2.0, jnp.float32)
    sbg = jnp.zeros(sx1.shape, jnp.float32)
    for g in range(_G):
        gx1 = gtv[:, 0:1, g:g + 1]
        gy1 = gtv[:, 1:2, g:g + 1]
        gx2 = gtv[:, 2:3, g:g + 1]
        gy2 = gtv[:, 3:4, g:g + 1]
        iw = jnp.clip(jnp.minimum(sx2, gx2) - jnp.maximum(sx1, gx1) + 1.0, 0.0)
        ih = jnp.clip(jnp.minimum(sy2, gy2) - jnp.maximum(sy1, gy1) + 1.0, 0.0)
        inter = iw * ih
        garea = (gx2 - gx1 + 1.0) * (gy2 - gy1 + 1.0)
        iou = inter / (sarea + garea - inter + 1e-6)
        val = jnp.where(jnp.float32(g) < nbv, iou, -1.0)
        upd = val > run_max
        run_max = jnp.where(upd, val, run_max)
        sbg = jnp.where(upd, jnp.float32(g), sbg)

    fg_row = sm >= 8.0
    sel_fg = jnp.logical_and(fg_row, lane < float(_KFG))

    zero_row = jnp.zeros((_B, 1, 128), jnp.float32)
    lab = zero_row
    gx1r = zero_row
    gy1r = zero_row
    gx2r = zero_row
    gy2r = zero_row
    for g in range(_G):
        hit = sbg == jnp.float32(g)
        lab = jnp.where(hit, gtv[:, 4:5, g:g + 1], lab)
        gx1r = jnp.where(hit, gtv[:, 0:1, g:g + 1], gx1r)
        gy1r = jnp.where(hit, gtv[:, 1:2, g:g + 1], gy1r)
        gx2r = jnp.where(hit, gtv[:, 2:3, g:g + 1], gx2r)
        gy2r = jnp.where(hit, gtv[:, 3:4, g:g + 1], gy2r)
    labels = jnp.where(sel_fg, lab, 0.0)

    ew = jnp.maximum(sx2 - sx1 + 1.0, 1e-6)
    eh = jnp.maximum(sy2 - sy1 + 1.0, 1e-6)
    ecx = sx1 + 0.5 * ew
    ecy = sy1 + 0.5 * eh
    gw = jnp.maximum(gx2r - gx1r + 1.0, 1e-6)
    gh = jnp.maximum(gy2r - gy1r + 1.0, 1e-6)
    gcx = gx1r + 0.5 * gw
    gcy = gy1r + 0.5 * gh
    dx = (gcx - ecx) / ew / _STD[0]
    dy = (gcy - ecy) / eh / _STD[1]
    dw = jnp.log(gw / ew) / _STD[2]
    dh = jnp.log(gh / eh) / _STD[3]
    fgf = jnp.where(sel_fg, 1.0, 0.0)
    dx = dx * fgf
    dy = dy * fgf
    dw = dw * fgf
    dh = dh * fgf

    infos = []
    for d in range(7):
        acc = zero_row
        for g in range(_G):
            acc = jnp.where(sbg == jnp.float32(g), infov[:, d:d + 1, g:g + 1],
                            acc)
        infos.append(acc)

    rows = [sx1, sy1, sx2, sy2, labels, fgf, dx, dy, dw, dh,
            gx1r, gy1r, gx2r, gy2r] + infos + [zero_row, zero_row, zero_row]
    out_ref[...] = jnp.concatenate(rows, axis=1)


@jax.jit
def kernel(all_rois, gt_boxes, num_boxes, gt_3d_info):
    B = all_rois.shape[0]
    coords = jnp.concatenate([all_rois[:, :, 1:5], gt_boxes[:, :, :4]], axis=1)
    coords = jnp.pad(coords, ((0, 0), (0, _NP - _NR), (0, 0)))
    coords = coords.transpose(0, 2, 1).reshape(B, 4, _ROWS, 128)

    gtv = jnp.pad(gt_boxes.transpose(0, 2, 1), ((0, 0), (0, 3), (0, 108)))
    nbv = jnp.broadcast_to(
        num_boxes.astype(jnp.float32)[:, None, None], (B, 1, 128))
    infov = jnp.pad(gt_3d_info.transpose(0, 2, 1), ((0, 0), (0, 1), (0, 108)))

    gtsc = jnp.concatenate(
        [jnp.pad(gt_boxes.transpose(0, 2, 1), ((0, 0), (0, 0), (0, 12))
                 ).reshape(B, 160),
         jnp.broadcast_to(
             num_boxes.astype(jnp.float32)[:, None], (B, 32))], axis=1)

    sel = _sc_select(coords.reshape(B, 4, _NP), gtsc)

    planes = pl.pallas_call(
        _tc2_body,
        grid=(1,),
        in_specs=[
            pl.BlockSpec((B, 4, _ROWS, 128), lambda b: (0, 0, 0, 0)),
            pl.BlockSpec((B, 1, 128), lambda b: (0, 0, 0)),
            pl.BlockSpec((B, 2, _K), lambda b: (0, 0, 0)),
            pl.BlockSpec((B, 8, 128), lambda b: (0, 0, 0)),
            pl.BlockSpec((B, 8, 128), lambda b: (0, 0, 0)),
        ],
        out_specs=pl.BlockSpec((B, 24, 128), lambda b: (0, 0, 0)),
        out_shape=jax.ShapeDtypeStruct((B, 24, 128), jnp.float32),
    )(coords, nbv, sel, gtv, infov)

    sx1 = planes[:, 0]
    sy1 = planes[:, 1]
    sx2 = planes[:, 2]
    sy2 = planes[:, 3]
    labels = planes[:, 4]
    fgf = planes[:, 5]
    rois = jnp.stack([jnp.zeros_like(sx1), sx1, sy1, sx2, sy2], axis=-1)
    bbox_targets = planes[:, 6:10].transpose(0, 2, 1)
    inside_w = jnp.broadcast_to(fgf[:, :, None], (B, _K, 4))
    outside_w = inside_w
    rois_for_3d = rois[:, :_KFG]
    gt_bbox_for_3d = planes[:, 10:14].transpose(0, 2, 1)[:, :_KFG]
    gt_3d_info_rois = planes[:, 14:21].transpose(0, 2, 1)[:, :_KFG]
    return (rois, labels, bbox_targets, inside_w, outside_w,
            rois_for_3d, gt_bbox_for_3d, gt_3d_info_rois)


# speedup vs baseline: 1.2361x; 1.2361x over previous
import functools

import jax
import jax.numpy as jnp
from jax import lax
from jax.experimental import pallas as pl
from jax.experimental.pallas import tpu as pltpu
from jax.experimental.pallas import tpu_sc as plsc

_N = 20000
_G = 20
_NR = _N + _G
_ROWS = 160
_NP = _ROWS * 128
_NCH = _NP // 16
_NL2 = _NCH // 16
_K = 128
_KFG = 32
_B = 4
_STD = (0.1, 0.1, 0.2, 0.2)
_LN2 = 0.6931471805599453


def _tc1_body(coords_ref, gt_ref, nb_ref, prio_ref):
    x1 = coords_ref[:, 0]
    y1 = coords_ref[:, 1]
    x2 = coords_ref[:, 2]
    y2 = coords_ref[:, 3]
    area = (x2 - x1 + 1.0) * (y2 - y1 + 1.0)

    gtv = gt_ref[...]
    nbv = nb_ref[...]

    run_max = jnp.full((_B, _ROWS, 128), -2.0, jnp.float32)
    for g in range(_G):
        gx1 = gtv[:, 0:1, g:g + 1]
        gy1 = gtv[:, 1:2, g:g + 1]
        gx2 = gtv[:, 2:3, g:g + 1]
        gy2 = gtv[:, 3:4, g:g + 1]
        iw = jnp.clip(jnp.minimum(x2, gx2) - jnp.maximum(x1, gx1) + 1.0, 0.0)
        ih = jnp.clip(jnp.minimum(y2, gy2) - jnp.maximum(y1, gy1) + 1.0, 0.0)
        inter = iw * ih
        garea = (gx2 - gx1 + 1.0) * (gy2 - gy1 + 1.0)
        iou = inter / (area + garea - inter + 1e-6)
        val = jnp.where(jnp.float32(g) < nbv, iou, -1.0)
        run_max = jnp.maximum(run_max, val)

    fg = run_max >= 0.5
    bgm = jnp.logical_and(run_max < 0.5, run_max >= 0.1)
    priority = run_max + jnp.where(fg, 10.0, 0.0) + jnp.where(bgm, 5.0, 0.0)

    gidx = (jax.lax.broadcasted_iota(jnp.int32, (_B, _ROWS, 128), 1) * 128
            + jax.lax.broadcasted_iota(jnp.int32, (_B, _ROWS, 128), 2)
            ).astype(jnp.float32)
    priority = jnp.where(gidx < float(_NR), priority, -1.0)

    prio_ref[...] = priority


def _first(mask, lane):
    return jnp.min(jnp.where(mask, lane, 10_000))


def _ln(x):
    bits = plsc.bitcast(x, jnp.int32)
    e = ((bits >> 23) & 0xFF) - 127
    m = plsc.bitcast((bits & 0x007FFFFF) | 0x3F800000, jnp.float32)
    y = (m - 1.0) / (m + 1.0)
    y2 = y * y
    p = y * (2.0 + y2 * (2.0 / 3.0 + y2 * (2.0 / 5.0 + y2 * (
        2.0 / 7.0 + y2 * (2.0 / 9.0)))))
    return e.astype(jnp.float32) * _LN2 + p


def _sc_body(prio_hbm, coords_hbm, gt_hbm, info_hbm, out_hbm,
             prio_v, x1_v, y1_v, x2_v, y2_v, gt_v, info_v, out_v,
             cmax_v, l2_v, keep_v, keepm_v):
    cid = lax.axis_index("c")
    sid = lax.axis_index("s")

    @pl.when(sid < 2)
    def _():
        b = cid * 2 + sid
        pltpu.sync_copy(prio_hbm.at[b], prio_v)
        pltpu.sync_copy(coords_hbm.at[b, 0], x1_v)
        pltpu.sync_copy(coords_hbm.at[b, 1], y1_v)
        pltpu.sync_copy(coords_hbm.at[b, 2], x2_v)
        pltpu.sync_copy(coords_hbm.at[b, 3], y2_v)
        pltpu.sync_copy(gt_hbm.at[b], gt_v)
        pltpu.sync_copy(info_hbm.at[b], info_v)
        lane = lax.broadcasted_iota(jnp.int32, (16,), 0)

        def build_cmax(k, _):
            acc = jnp.full((16,), -9.0, jnp.float32)
            for j in range(16):
                v = prio_v[pl.ds((k * 16 + j) * 16, 16)]
                acc = jnp.where(lane == j, jnp.max(v), acc)
            cmax_v[pl.ds(k * 16, 16)] = acc
            return 0
        lax.fori_loop(0, _NL2, build_cmax, 0)

        def build_l2(k, _):
            acc = jnp.full((16,), -9.0, jnp.float32)
            for j in range(16):
                v = cmax_v[pl.ds((k * 16 + j) * 16, 16)]
                acc = jnp.where(lane == j, jnp.max(v), acc)
            l2_v[pl.ds(k * 16, 16)] = acc
            return 0
        lax.fori_loop(0, _NL2 // 16, build_l2, 0)

        l3 = jnp.full((16,), -9.0, jnp.float32)
        for h in range(_NL2 // 16):
            v = l2_v[pl.ds(h * 16, 16)]
            l3 = jnp.where(lane == h, jnp.max(v), l3)

        def outer(o, l3):
            ki = jnp.zeros((16,), jnp.float32)
            km = jnp.zeros((16,), jnp.float32)
            for j in range(16):
                m = jnp.max(l3)
                h = _first(l3 == m, lane)
                l2v = l2_v[pl.ds(h * 16, 16)]
                s2 = h * 16 + _first(l2v == m, lane)
                cmv = cmax_v[pl.ds(s2 * 16, 16)]
                s3 = s2 * 16 + _first(cmv == m, lane)
                pv = prio_v[pl.ds(s3 * 16, 16)]
                s4 = _first(pv == m, lane)
                idx = s3 * 16 + s4
                pv = jnp.where(lane == s4, -3.0, pv)
                prio_v[pl.ds(s3 * 16, 16)] = pv
                cmv = jnp.where(lane == (s3 - s2 * 16), jnp.max(pv), cmv)
                cmax_v[pl.ds(s2 * 16, 16)] = cmv
                l2v = jnp.where(lane == (s2 - h * 16), jnp.max(cmv), l2v)
                l2_v[pl.ds(h * 16, 16)] = l2v
                l3 = jnp.where(lane == h, jnp.max(l2v), l3)
                ki = jnp.where(lane == j, idx.astype(jnp.float32), ki)
                km = jnp.where(lane == j, m, km)
            keep_v[pl.ds(o * 16, 16)] = ki
            keepm_v[pl.ds(o * 16, 16)] = km
            return l3
        lax.fori_loop(0, _K // 16, outer, l3)

        def _sca(row, g):
            base = row * 32 + (16 if g >= 16 else 0)
            v = gt_v[pl.ds(base, 16)]
            return jnp.sum(jnp.where(lane == (g % 16), v, 0.0))

        nb = _sca(5, 0)
        gxs = [(_sca(0, g), _sca(1, g), _sca(2, g), _sca(3, g))
               for g in range(_G)]

        for o in range(_K // 16):
            ki = keep_v[pl.ds(o * 16, 16)].astype(jnp.int32)
            km = keepm_v[pl.ds(o * 16, 16)]
            sx1 = plsc.load_gather(x1_v, [ki])
            sy1 = plsc.load_gather(y1_v, [ki])
            sx2 = plsc.load_gather(x2_v, [ki])
            sy2 = plsc.load_gather(y2_v, [ki])

            sarea = (sx2 - sx1 + 1.0) * (sy2 - sy1 + 1.0)
            run_max = jnp.full((16,), -2.0, jnp.float32)
            sbg = jnp.zeros((16,), jnp.float32)
            for g in range(_G):
                gx1, gy1, gx2, gy2 = gxs[g]
                iw = jnp.clip(
                    jnp.minimum(sx2, gx2) - jnp.maximum(sx1, gx1) + 1.0, 0.0)
                ih = jnp.clip(
                    jnp.minimum(sy2, gy2) - jnp.maximum(sy1, gy1) + 1.0, 0.0)
                inter = iw * ih
                garea = (gx2 - gx1 + 1.0) * (gy2 - gy1 + 1.0)
                iou = inter / (sarea + garea - inter + 1e-6)
                val = jnp.where(jnp.float32(g) < nb, iou, -1.0)
                upd = val > run_max
                run_max = jnp.where(upd, val, run_max)
                sbg = jnp.where(upd, jnp.float32(g), sbg)

            sbgi = sbg.astype(jnp.int32)
            gx1r = plsc.load_gather(gt_v, [sbgi])
            gy1r = plsc.load_gather(gt_v, [sbgi + 32])
            gx2r = plsc.load_gather(gt_v, [sbgi + 64])
            gy2r = plsc.load_gather(gt_v, [sbgi + 96])
            lab = plsc.load_gather(gt_v, [sbgi + 128])

            pos = o * 16 + lane
            sel_fg = jnp.logical_and(km >= 8.0, pos < _KFG)
            fgf = jnp.where(sel_fg, 1.0, 0.0)
            labels = jnp.where(sel_fg, lab, 0.0)

            ew = jnp.maximum(sx2 - sx1 + 1.0, 1e-6)
            eh = jnp.maximum(sy2 - sy1 + 1.0, 1e-6)
            ecx = sx1 + 0.5 * ew
            ecy = sy1 + 0.5 * eh
            gw = jnp.maximum(gx2r - gx1r + 1.0, 1e-6)
            gh = jnp.maximum(gy2r - gy1r + 1.0, 1e-6)
            gcx = gx1r + 0.5 * gw
            gcy = gy1r + 0.5 * gh
            dx = (gcx - ecx) / ew * (1.0 / _STD[0]) * fgf
            dy = (gcy - ecy) / eh * (1.0 / _STD[1]) * fgf
            dw = _ln(gw / ew) * (1.0 / _STD[2]) * fgf
            dh = _ln(gh / eh) * (1.0 / _STD[3]) * fgf

            rows = [sx1, sy1, sx2, sy2, labels, fgf, dx, dy, dw, dh,
                    gx1r, gy1r, gx2r, gy2r]
            for d in range(7):
                rows.append(plsc.load_gather(info_v, [sbgi * 8 + d]))
            zeros = jnp.zeros((16,), jnp.float32)
            rows += [zeros, zeros, zeros]
            for r, vec in enumerate(rows):
                out_v[pl.ds(r * 128 + o * 16, 16)] = vec

        pltpu.sync_copy(out_v, out_hbm.at[b])


_sc_select = functools.partial(
    pl.kernel,
    out_type=jax.ShapeDtypeStruct((_B, 24 * 128), jnp.float32),
    mesh=plsc.VectorSubcoreMesh(core_axis_name="c", subcore_axis_name="s"),
    scratch_types=[
        pltpu.VMEM((_NP,), jnp.float32),
        pltpu.VMEM((_NP,), jnp.float32),
        pltpu.VMEM((_NP,), jnp.float32),
        pltpu.VMEM((_NP,), jnp.float32),
        pltpu.VMEM((_NP,), jnp.float32),
        pltpu.VMEM((192,), jnp.float32),
        pltpu.VMEM((_G * 8,), jnp.float32),
        pltpu.VMEM((24 * 128,), jnp.float32),
        pltpu.VMEM((_NCH,), jnp.float32),
        pltpu.VMEM((_NL2,), jnp.float32),
        pltpu.VMEM((_K,), jnp.float32),
        pltpu.VMEM((_K,), jnp.float32),
    ],
    compiler_params=pltpu.CompilerParams(needs_layout_passes=False),
)(_sc_body)


@jax.jit
def kernel(all_rois, gt_boxes, num_boxes, gt_3d_info):
    B = all_rois.shape[0]
    coords = jnp.concatenate([all_rois[:, :, 1:5], gt_boxes[:, :, :4]], axis=1)
    coords = jnp.pad(coords, ((0, 0), (0, _NP - _NR), (0, 0)))
    coords = coords.transpose(0, 2, 1).reshape(B, 4, _ROWS, 128)

    gtv = jnp.pad(gt_boxes.transpose(0, 2, 1), ((0, 0), (0, 3), (0, 108)))
    nbv = jnp.broadcast_to(
        num_boxes.astype(jnp.float32)[:, None, None], (B, 1, 128))

    prio = pl.pallas_call(
        _tc1_body,
        grid=(1,),
        in_specs=[
            pl.BlockSpec((B, 4, _ROWS, 128), lambda b: (0, 0, 0, 0)),
            pl.BlockSpec((B, 8, 128), lambda b: (0, 0, 0)),
            pl.BlockSpec((B, 1, 128), lambda b: (0, 0, 0)),
        ],
        out_specs=pl.BlockSpec((B, _ROWS, 128), lambda b: (0, 0, 0)),
        out_shape=jax.ShapeDtypeStruct((B, _ROWS, 128), jnp.float32),
    )(coords, gtv, nbv)

    gtsc = jnp.concatenate(
        [jnp.pad(gt_boxes.transpose(0, 2, 1), ((0, 0), (0, 0), (0, 12))
                 ).reshape(B, 160),
         jnp.broadcast_to(num_boxes.astype(jnp.float32)[:, None], (B, 32))],
        axis=1)
    infosc = jnp.pad(gt_3d_info, ((0, 0), (0, 0), (0, 1))).reshape(B, _G * 8)

    planes = _sc_select(prio.reshape(B, _NP), coords.reshape(B, 4, _NP),
                        gtsc, infosc).reshape(B, 24, 128)

    sx1 = planes[:, 0]
    sy1 = planes[:, 1]
    sx2 = planes[:, 2]
    sy2 = planes[:, 3]
    labels = planes[:, 4]
    fgf = planes[:, 5]
    rois = jnp.stack([jnp.zeros_like(sx1), sx1, sy1, sx2, sy2], axis=-1)
    bbox_targets = planes[:, 6:10].transpose(0, 2, 1)
    inside_w = jnp.broadcast_to(fgf[:, :, None], (B, _K, 4))
    outside_w = inside_w
    rois_for_3d = rois[:, :_KFG]
    gt_bbox_for_3d = planes[:, 10:14].transpose(0, 2, 1)[:, :_KFG]
    gt_3d_info_rois = planes[:, 14:21].transpose(0, 2, 1)[:, :_KFG]
    return (rois, labels, bbox_targets, inside_w, outside_w,
            rois_for_3d, gt_bbox_for_3d, gt_3d_info_rois)


# speedup vs baseline: 1.4785x; 1.1961x over previous
import functools

import jax
import jax.numpy as jnp
from jax import lax
from jax.experimental import pallas as pl
from jax.experimental.pallas import tpu as pltpu
from jax.experimental.pallas import tpu_sc as plsc

_N = 20000
_G = 20
_NR = _N + _G
_ROWS = 160
_NP = _ROWS * 128
_NCH = _NP // 16
_NL2 = _NCH // 16
_K = 128
_KFG = 32
_B = 4
_STD = (0.1, 0.1, 0.2, 0.2)


def _tc1_body(coords_ref, gt_ref, nb_ref, prio_ref):
    x1 = coords_ref[:, 0]
    y1 = coords_ref[:, 1]
    x2 = coords_ref[:, 2]
    y2 = coords_ref[:, 3]
    area = (x2 - x1 + 1.0) * (y2 - y1 + 1.0)

    gtv = gt_ref[...]
    nbv = nb_ref[...]

    run_max = jnp.full((_B, _ROWS, 128), -2.0, jnp.float32)
    for g in range(_G):
        gx1 = gtv[:, 0:1, g:g + 1]
        gy1 = gtv[:, 1:2, g:g + 1]
        gx2 = gtv[:, 2:3, g:g + 1]
        gy2 = gtv[:, 3:4, g:g + 1]
        iw = jnp.clip(jnp.minimum(x2, gx2) - jnp.maximum(x1, gx1) + 1.0, 0.0)
        ih = jnp.clip(jnp.minimum(y2, gy2) - jnp.maximum(y1, gy1) + 1.0, 0.0)
        inter = iw * ih
        garea = (gx2 - gx1 + 1.0) * (gy2 - gy1 + 1.0)
        iou = inter / (area + garea - inter + 1e-6)
        val = jnp.where(jnp.float32(g) < nbv, iou, -1.0)
        run_max = jnp.maximum(run_max, val)

    fg = run_max >= 0.5
    bgm = jnp.logical_and(run_max < 0.5, run_max >= 0.1)
    priority = run_max + jnp.where(fg, 10.0, 0.0) + jnp.where(bgm, 5.0, 0.0)

    gidx = (jax.lax.broadcasted_iota(jnp.int32, (_B, _ROWS, 128), 1) * 128
            + jax.lax.broadcasted_iota(jnp.int32, (_B, _ROWS, 128), 2)
            ).astype(jnp.float32)
    priority = jnp.where(gidx < float(_NR), priority, -1.0)

    prio_ref[...] = priority


def _first(mask, lane):
    return jnp.min(jnp.where(mask, lane, 10_000))


def _sc_body(prio_hbm, out_hbm, prio_v, cmax_v, l2_v, keep_v, keepm_v):
    cid = lax.axis_index("c")
    sid = lax.axis_index("s")

    @pl.when(sid < 2)
    def _():
        b = cid * 2 + sid
        pltpu.sync_copy(prio_hbm.at[b], prio_v)
        lane = lax.broadcasted_iota(jnp.int32, (16,), 0)

        def build_cmax(k, _):
            acc = jnp.full((16,), -9.0, jnp.float32)
            for j in range(16):
                v = prio_v[pl.ds((k * 16 + j) * 16, 16)]
                acc = jnp.where(lane == j, jnp.max(v), acc)
            cmax_v[pl.ds(k * 16, 16)] = acc
            return 0
        lax.fori_loop(0, _NL2, build_cmax, 0)

        def build_l2(k, _):
            acc = jnp.full((16,), -9.0, jnp.float32)
            for j in range(16):
                v = cmax_v[pl.ds((k * 16 + j) * 16, 16)]
                acc = jnp.where(lane == j, jnp.max(v), acc)
            l2_v[pl.ds(k * 16, 16)] = acc
            return 0
        lax.fori_loop(0, _NL2 // 16, build_l2, 0)

        l3 = jnp.full((16,), -9.0, jnp.float32)
        for h in range(_NL2 // 16):
            v = l2_v[pl.ds(h * 16, 16)]
            l3 = jnp.where(lane == h, jnp.max(v), l3)

        def outer(o, l3):
            ki = jnp.zeros((16,), jnp.float32)
            km = jnp.zeros((16,), jnp.float32)
            for j in range(16):
                m = jnp.max(l3)
                h = _first(l3 == m, lane)
                l2v = l2_v[pl.ds(h * 16, 16)]
                s2 = h * 16 + _first(l2v == m, lane)
                cmv = cmax_v[pl.ds(s2 * 16, 16)]
                s3 = s2 * 16 + _first(cmv == m, lane)
                pv = prio_v[pl.ds(s3 * 16, 16)]
                s4 = _first(pv == m, lane)
                idx = s3 * 16 + s4
                pv = jnp.where(lane == s4, -3.0, pv)
                prio_v[pl.ds(s3 * 16, 16)] = pv
                cmv = jnp.where(lane == (s3 - s2 * 16), jnp.max(pv), cmv)
                cmax_v[pl.ds(s2 * 16, 16)] = cmv
                l2v = jnp.where(lane == (s2 - h * 16), jnp.max(cmv), l2v)
                l2_v[pl.ds(h * 16, 16)] = l2v
                l3 = jnp.where(lane == h, jnp.max(l2v), l3)
                ki = jnp.where(lane == j, idx.astype(jnp.float32), ki)
                km = jnp.where(lane == j, m, km)
            keep_v[pl.ds(o * 16, 16)] = ki
            keepm_v[pl.ds(o * 16, 16)] = km
            return l3
        lax.fori_loop(0, _K // 16, outer, l3)

        pltpu.sync_copy(keep_v, out_hbm.at[b, 0])
        pltpu.sync_copy(keepm_v, out_hbm.at[b, 1])


_sc_select = functools.partial(
    pl.kernel,
    out_type=jax.ShapeDtypeStruct((_B, 2, _K), jnp.float32),
    mesh=plsc.VectorSubcoreMesh(core_axis_name="c", subcore_axis_name="s"),
    scratch_types=[
        pltpu.VMEM((_NP,), jnp.float32),
        pltpu.VMEM((_NCH,), jnp.float32),
        pltpu.VMEM((_NL2,), jnp.float32),
        pltpu.VMEM((_K,), jnp.float32),
        pltpu.VMEM((_K,), jnp.float32),
    ],
    compiler_params=pltpu.CompilerParams(needs_layout_passes=False),
)(_sc_body)


def _tc2_body(coords_ref, nb_ref, sel_ref, gt_ref, info_ref, out_ref):
    x1 = coords_ref[:, 0]
    y1 = coords_ref[:, 1]
    x2 = coords_ref[:, 2]
    y2 = coords_ref[:, 3]
    nbv = nb_ref[...]
    gtv = gt_ref[...]
    infov = info_ref[...]

    idx = sel_ref[:, 0:1, :]
    sm = sel_ref[:, 1:2, :]
    kr = jnp.floor(idx * (1.0 / 128.0))
    kc = idx - 128.0 * kr

    lane = jax.lax.broadcasted_iota(
        jnp.int32, (1, 1, 128), 2).astype(jnp.float32)

    e_c = jnp.where(
        jax.lax.broadcasted_iota(
            jnp.int32, (_B, 128, 128), 1).astype(jnp.float32) == kc,
        1.0, 0.0)
    e_r = jnp.where(
        jax.lax.broadcasted_iota(
            jnp.int32, (_B, _ROWS, 128), 1).astype(jnp.float32) == kr,
        1.0, 0.0)

    def pick(q):
        y = jax.lax.dot_general(
            q, e_c, dimension_numbers=(((2,), (1,)), ((0,), (0,))),
            precision=jax.lax.Precision.HIGHEST)
        return jnp.sum(e_r * y, axis=1, keepdims=True)

    sx1 = pick(x1)
    sy1 = pick(y1)
    sx2 = pick(x2)
    sy2 = pick(y2)

    sarea = (sx2 - sx1 + 1.0) * (sy2 - sy1 + 1.0)
    run_max = jnp.full(sx1.shape, -2.0, jnp.float32)
    sbg = jnp.zeros(sx1.shape, jnp.float32)
    for g in range(_G):
        gx1 = gtv[:, 0:1, g:g + 1]
        gy1 = gtv[:, 1:2, g:g + 1]
        gx2 = gtv[:, 2:3, g:g + 1]
        gy2 = gtv[:, 3:4, g:g + 1]
        iw = jnp.clip(jnp.minimum(sx2, gx2) - jnp.maximum(sx1, gx1) + 1.0, 0.0)
        ih = jnp.clip(jnp.minimum(sy2, gy2) - jnp.maximum(sy1, gy1) + 1.0, 0.0)
        inter = iw * ih
        garea = (gx2 - gx1 + 1.0) * (gy2 - gy1 + 1.0)
        iou = inter / (sarea + garea - inter + 1e-6)
        val = jnp.where(jnp.float32(g) < nbv, iou, -1.0)
        upd = val > run_max
        run_max = jnp.where(upd, val, run_max)
        sbg = jnp.where(upd, jnp.float32(g), sbg)

    fg_row = sm >= 8.0
    sel_fg = jnp.logical_and(fg_row, lane < float(_KFG))

    zero_row = jnp.zeros((_B, 1, 128), jnp.float32)
    lab = zero_row
    gx1r = zero_row
    gy1r = zero_row
    gx2r = zero_row
    gy2r = zero_row
    for g in range(_G):
        hit = sbg == jnp.float32(g)
        lab = jnp.where(hit, gtv[:, 4:5, g:g + 1], lab)
        gx1r = jnp.where(hit, gtv[:, 0:1, g:g + 1], gx1r)
        gy1r = jnp.where(hit, gtv[:, 1:2, g:g + 1], gy1r)
        gx2r = jnp.where(hit, gtv[:, 2:3, g:g + 1], gx2r)
        gy2r = jnp.where(hit, gtv[:, 3:4, g:g + 1], gy2r)
    labels = jnp.where(sel_fg, lab, 0.0)

    ew = jnp.maximum(sx2 - sx1 + 1.0, 1e-6)
    eh = jnp.maximum(sy2 - sy1 + 1.0, 1e-6)
    ecx = sx1 + 0.5 * ew
    ecy = sy1 + 0.5 * eh
    gw = jnp.maximum(gx2r - gx1r + 1.0, 1e-6)
    gh = jnp.maximum(gy2r - gy1r + 1.0, 1e-6)
    gcx = gx1r + 0.5 * gw
    gcy = gy1r + 0.5 * gh
    dx = (gcx - ecx) / ew / _STD[0]
    dy = (gcy - ecy) / eh / _STD[1]
    dw = jnp.log(gw / ew) / _STD[2]
    dh = jnp.log(gh / eh) / _STD[3]
    fgf = jnp.where(sel_fg, 1.0, 0.0)
    dx = dx * fgf
    dy = dy * fgf
    dw = dw * fgf
    dh = dh * fgf

    infos = []
    for d in range(7):
        acc = zero_row
        for g in range(_G):
            acc = jnp.where(sbg == jnp.float32(g), infov[:, d:d + 1, g:g + 1],
                            acc)
        infos.append(acc)

    rows = [sx1, sy1, sx2, sy2, labels, fgf, dx, dy, dw, dh,
            gx1r, gy1r, gx2r, gy2r] + infos + [zero_row, zero_row, zero_row]
    out_ref[...] = jnp.concatenate(rows, axis=1)


@jax.jit
def kernel(all_rois, gt_boxes, num_boxes, gt_3d_info):
    B = all_rois.shape[0]
    coords = jnp.concatenate([all_rois[:, :, 1:5], gt_boxes[:, :, :4]], axis=1)
    coords = jnp.pad(coords, ((0, 0), (0, _NP - _NR), (0, 0)))
    coords = coords.transpose(0, 2, 1).reshape(B, 4, _ROWS, 128)

    gtv = jnp.pad(gt_boxes.transpose(0, 2, 1), ((0, 0), (0, 3), (0, 108)))
    nbv = jnp.broadcast_to(
        num_boxes.astype(jnp.float32)[:, None, None], (B, 1, 128))
    infov = jnp.pad(gt_3d_info.transpose(0, 2, 1), ((0, 0), (0, 1), (0, 108)))

    prio = pl.pallas_call(
        _tc1_body,
        grid=(1,),
        in_specs=[
            pl.BlockSpec((B, 4, _ROWS, 128), lambda b: (0, 0, 0, 0)),
            pl.BlockSpec((B, 8, 128), lambda b: (0, 0, 0)),
            pl.BlockSpec((B, 1, 128), lambda b: (0, 0, 0)),
        ],
        out_specs=pl.BlockSpec((B, _ROWS, 128), lambda b: (0, 0, 0)),
        out_shape=jax.ShapeDtypeStruct((B, _ROWS, 128), jnp.float32),
    )(coords, gtv, nbv)

    sel = _sc_select(prio.reshape(B, _NP))

    planes = pl.pallas_call(
        _tc2_body,
        grid=(1,),
        in_specs=[
            pl.BlockSpec((B, 4, _ROWS, 128), lambda b: (0, 0, 0, 0)),
            pl.BlockSpec((B, 1, 128), lambda b: (0, 0, 0)),
            pl.BlockSpec((B, 2, _K), lambda b: (0, 0, 0)),
            pl.BlockSpec((B, 8, 128), lambda b: (0, 0, 0)),
            pl.BlockSpec((B, 8, 128), lambda b: (0, 0, 0)),
        ],
        out_specs=pl.BlockSpec((B, 24, 128), lambda b: (0, 0, 0)),
        out_shape=jax.ShapeDtypeStruct((B, 24, 128), jnp.float32),
    )(coords, nbv, sel, gtv, infov)

    sx1 = planes[:, 0]
    sy1 = planes[:, 1]
    sx2 = planes[:, 2]
    sy2 = planes[:, 3]
    labels = planes[:, 4]
    fgf = planes[:, 5]
    rois = jnp.stack([jnp.zeros_like(sx1), sx1, sy1, sx2, sy2], axis=-1)
    bbox_targets = planes[:, 6:10].transpose(0, 2, 1)
    inside_w = jnp.broadcast_to(fgf[:, :, None], (B, _K, 4))
    outside_w = inside_w
    rois_for_3d = rois[:, :_KFG]
    gt_bbox_for_3d = planes[:, 10:14].transpose(0, 2, 1)[:, :_KFG]
    gt_3d_info_rois = planes[:, 14:21].transpose(0, 2, 1)[:, :_KFG]
    return (rois, labels, bbox_targets, inside_w, outside_w,
            rois_for_3d, gt_bbox_for_3d, gt_3d_info_rois)
